# speedup vs baseline: 1.0804x; 1.0804x over previous
import functools

import jax
import jax.numpy as jnp
from jax.experimental import pallas as pl

N0, N1, NNET = 10000, 2500, 1000


def _gat_mm_body(x_ref, w_ref, al_ref, ar_ref, z_ref, el_ref, er_ref):
    z = jnp.dot(x_ref[...], w_ref[...], preferred_element_type=jnp.float32)
    z_ref[...] = z
    el_ref[...] = jnp.sum(z * al_ref[...], axis=1, keepdims=True)
    er_ref[...] = jnp.sum(z * ar_ref[...], axis=1, keepdims=True)


def _gat_mm(x, W, al, ar, tn=512):
    n, k = x.shape
    g = W.shape[1]
    grid = (pl.cdiv(n, tn),)
    z, el, er = pl.pallas_call(
        _gat_mm_body,
        grid=grid,
        in_specs=[
            pl.BlockSpec((tn, k), lambda i: (i, 0)),
            pl.BlockSpec((k, g), lambda i: (0, 0)),
            pl.BlockSpec((1, g), lambda i: (0, 0)),
            pl.BlockSpec((1, g), lambda i: (0, 0)),
        ],
        out_specs=[
            pl.BlockSpec((tn, g), lambda i: (i, 0)),
            pl.BlockSpec((tn, 1), lambda i: (i, 0)),
            pl.BlockSpec((tn, 1), lambda i: (i, 0)),
        ],
        out_shape=[
            jax.ShapeDtypeStruct((n, g), jnp.float32),
            jax.ShapeDtypeStruct((n, 1), jnp.float32),
            jax.ShapeDtypeStruct((n, 1), jnp.float32),
        ],
    )(x, W, al.reshape(1, g), ar.reshape(1, g))
    return z, el[:, 0], er[:, 0]


def _gat_mm2_body(s_ref, x0_ref, cnt_ref, wa_ref, wb_ref, al_ref, ar_ref,
                  z_ref, el_ref, er_ref):
    z = jnp.dot(s_ref[...], wa_ref[...], preferred_element_type=jnp.float32)
    z = z + jnp.dot(x0_ref[...] * cnt_ref[...], wb_ref[...],
                    preferred_element_type=jnp.float32)
    z_ref[...] = z
    el_ref[...] = jnp.sum(z * al_ref[...], axis=1, keepdims=True)
    er_ref[...] = jnp.sum(z * ar_ref[...], axis=1, keepdims=True)


def _gat_mm2(s, x0, cnt, Wa, Wb, al, ar, tn=512):
    n, k = s.shape
    g = Wa.shape[1]
    grid = (pl.cdiv(n, tn),)
    z, el, er = pl.pallas_call(
        _gat_mm2_body,
        grid=grid,
        in_specs=[
            pl.BlockSpec((tn, k), lambda i: (i, 0)),
            pl.BlockSpec((tn, k), lambda i: (i, 0)),
            pl.BlockSpec((tn, 1), lambda i: (i, 0)),
            pl.BlockSpec((k, g), lambda i: (0, 0)),
            pl.BlockSpec((k, g), lambda i: (0, 0)),
            pl.BlockSpec((1, g), lambda i: (0, 0)),
            pl.BlockSpec((1, g), lambda i: (0, 0)),
        ],
        out_specs=[
            pl.BlockSpec((tn, g), lambda i: (i, 0)),
            pl.BlockSpec((tn, 1), lambda i: (i, 0)),
            pl.BlockSpec((tn, 1), lambda i: (i, 0)),
        ],
        out_shape=[
            jax.ShapeDtypeStruct((n, g), jnp.float32),
            jax.ShapeDtypeStruct((n, 1), jnp.float32),
            jax.ShapeDtypeStruct((n, 1), jnp.float32),
        ],
    )(s, x0, cnt.reshape(n, 1), Wa, Wb, al.reshape(1, g), ar.reshape(1, g))
    return z, el[:, 0], er[:, 0]


def _head_body(y_ref, w0_ref, b0_ref, w1_ref, b1_ref, o_ref):
    h = jnp.dot(y_ref[...], w0_ref[...], preferred_element_type=jnp.float32)
    h = jnp.maximum(h + b0_ref[...], 0.0)
    o = jnp.dot(h, w1_ref[...], preferred_element_type=jnp.float32)
    o_ref[...] = jnp.maximum(o + b1_ref[...], 0.0)


def _head(y, M0W, M0b, M1W, M1b):
    n, k = y.shape
    h1 = M0W.shape[1]
    h2 = M1W.shape[1]
    return pl.pallas_call(
        _head_body,
        grid=(1,),
        in_specs=[
            pl.BlockSpec((n, k), lambda i: (0, 0)),
            pl.BlockSpec((k, h1), lambda i: (0, 0)),
            pl.BlockSpec((1, h1), lambda i: (0, 0)),
            pl.BlockSpec((h1, h2), lambda i: (0, 0)),
            pl.BlockSpec((1, h2), lambda i: (0, 0)),
        ],
        out_specs=pl.BlockSpec((n, h2), lambda i: (0, 0)),
        out_shape=jax.ShapeDtypeStruct((n, h2), jnp.float32),
    )(y, M0W, M0b.reshape(1, h1), M1W, M1b.reshape(1, h2))


def _gat_edges(z, el, er, src, dst, b, n_dst):
    e = el[src] + er[dst]
    e = jnp.where(e > 0, e, 0.2 * e)
    m = jax.ops.segment_max(e, dst, num_segments=n_dst)
    m = jnp.where(jnp.isfinite(m), m, 0.0)
    ex = jnp.exp(e - m[dst])
    den = jax.ops.segment_sum(ex, dst, num_segments=n_dst)
    den = jnp.where(den == 0.0, 1.0, den)
    alpha = ex / den[dst]
    out = jax.ops.segment_sum(alpha[:, None] * z[src], dst, num_segments=n_dst)
    return out + b


def kernel(X, e00_src, e00_dst, down_src, down_dst, e11_src, e11_dst,
           up_src, up_dst, conn_src, conn_dst, W0, al0, ar0, b0,
           W1, al1, ar1, b1, W2, al2, ar2, b2, M0W, M0b, M1W, M1b):
    z0, el0, er0 = _gat_mm(X, W0, al0, ar0)
    x0 = _gat_edges(z0, el0, er0, e00_src, e00_dst, b0, N0)

    s = jax.ops.segment_sum(x0[down_src], down_dst, num_segments=N1)
    c = jax.ops.segment_sum(jnp.ones_like(down_dst, jnp.float32), down_dst,
                            num_segments=N1)
    x1 = jnp.maximum(s / jnp.maximum(c, 1.0)[:, None], 0.0)

    z1, el1, er1 = _gat_mm(x1, W1, al1, ar1)
    x1 = jnp.maximum(_gat_edges(z1, el1, er1, e11_src, e11_dst, b1, N1), 0.0)

    up_sum = jax.ops.segment_sum(x1[up_src], up_dst, num_segments=N0)
    up_cnt = jax.ops.segment_sum(jnp.ones_like(up_dst, jnp.float32), up_dst,
                                 num_segments=N0)
    z2, el2, er2 = _gat_mm2(up_sum, x0, up_cnt, W2[:256], W2[256:], al2, ar2)
    x0u = jnp.maximum(_gat_edges(z2, el2, er2, e00_src, e00_dst, b2, N0), 0.0)

    y = jax.ops.segment_sum(x0u[conn_src], conn_dst, num_segments=NNET)

    return _head(y, M0W, M0b, M1W, M1b)


# speedup vs baseline: 12.1781x; 11.2720x over previous
import functools

import jax
import jax.numpy as jnp
from jax import lax
from jax.experimental import pallas as pl
from jax.experimental.pallas import tpu as pltpu
from jax.experimental.pallas import tpu_sc as plsc

N0, N1, NNET = 10000, 2500, 1000
NT = 16
CHUNK = 128
RB = 32


def _cdiv(a, b):
    return (a + b - 1) // b


@functools.lru_cache(maxsize=None)
def _sc_seg(n_off, n_dst, e_pad, softmax, divide, relu, bias, want_cnt):
    npad = 1024 * _cdiv(n_dst + 1, 1024)
    ep = e_pad // NT
    nct = ep // CHUNK
    npr = npad // NT
    nblk = npr // RB
    need_den = softmax or divide or want_cnt

    mesh = plsc.VectorSubcoreMesh(core_axis_name="c", subcore_axis_name="s")

    out_type = [jax.ShapeDtypeStruct((2 * npad, 128), jnp.float32)]
    if want_cnt:
        out_type.append(jax.ShapeDtypeStruct((npad,), jnp.float32))

    scratch = [
        pltpu.VMEM((CHUNK,), jnp.int32),
        pltpu.VMEM((CHUNK,), jnp.int32),
        pltpu.VMEM((CHUNK, 128), jnp.float32),
        pltpu.VMEM((RB, 128), jnp.float32),
        pltpu.VMEM_SHARED((npad, 128), jnp.float32),
        pltpu.SemaphoreType.DMA,
    ]
    if softmax:
        scratch += [
            pltpu.VMEM((CHUNK,), jnp.float32),
            pltpu.VMEM((npad,), jnp.float32),
            pltpu.VMEM((npad,), jnp.float32),
            pltpu.VMEM_SHARED((npad,), jnp.float32),
            pltpu.VMEM_SHARED((npad,), jnp.float32),
        ]
    if bias:
        scratch.append(pltpu.VMEM((128,), jnp.float32))
    if need_den:
        scratch += [
            pltpu.VMEM((CHUNK,), jnp.float32),
            pltpu.VMEM_SHARED((npad,), jnp.float32),
            pltpu.VMEM((RB,), jnp.float32),
        ]
    if want_cnt:
        scratch.append(pltpu.VMEM((RB,), jnp.float32))

    def body(*refs):
        refs = list(refs)
        src_r, dst_r, ztab_r = refs[:3]
        refs = refs[3:]
        el_r = refs.pop(0) if softmax else None
        er_r = refs.pop(0) if softmax else None
        b_r = refs.pop(0) if bias else None
        out_r = refs.pop(0)
        cnt_r = refs.pop(0) if want_cnt else None
        src_c, dst_c, gbuf, outblk, accF, sem = refs[:6]
        refs = refs[6:]
        if softmax:
            exbuf, el_t, er_t, el_sh, er_sh = refs[:5]
            refs = refs[5:]
        if bias:
            bvec = refs.pop(0)
        if need_den:
            exd, accD, denblk = refs[:3]
            refs = refs[3:]
        if want_cnt:
            cntbuf = refs.pop(0)

        c = lax.axis_index("c")
        s = lax.axis_index("s")

        if softmax:
            @pl.when(s == 0)
            def _():
                pltpu.sync_copy(el_r, el_sh)
                pltpu.sync_copy(er_r, er_sh)
        if bias:
            pltpu.sync_copy(b_r.at[c], bvec)

        zv = jnp.zeros((16,), jnp.float32)
        for r in range(RB):
            for k in range(8):
                outblk[r, pl.ds(k * 16, 16)] = zv
        if need_den:
            for r in range(RB // 16):
                denblk[pl.ds(r * 16, 16)] = zv
        t0 = s * npr

        def zstp(b, carry):
            zrow = pl.multiple_of(t0 + b * RB, RB)
            pltpu.sync_copy(outblk, accF.at[pl.ds(zrow, RB)])
            if need_den:
                pltpu.sync_copy(denblk, accD.at[pl.ds(zrow, RB)])
            return carry
        lax.fori_loop(0, nblk, zstp, 0)

        if need_den and not softmax:
            onev = jnp.ones((16,), jnp.float32)
            for r in range(CHUNK // 16):
                exd[pl.ds(r * 16, 16)] = onev

        plsc.subcore_barrier()

        if softmax:
            pltpu.sync_copy(el_sh, el_t)
            pltpu.sync_copy(er_sh, er_t)

            def _splatmax(vec):
                exbuf[pl.ds(0, 16)] = vec
                acc = plsc.load_gather(exbuf, [jnp.zeros((16,), jnp.int32)])
                for i in range(1, 16):
                    acc = jnp.maximum(
                        acc, plsc.load_gather(
                            exbuf, [jnp.full((16,), i, jnp.int32)]))
                return acc

            def _tabmax(ref):
                def stp(i, m):
                    return jnp.maximum(m, ref[pl.ds(i * 16, 16)])
                m0 = jnp.full((16,), -3.4e38, jnp.float32)
                return _splatmax(lax.fori_loop(0, npad // 16, stp, m0))
            ms = _tabmax(el_t) + _tabmax(er_t)
            mglob = jnp.where(ms > 0, ms, 0.2 * ms)

        coff = c * n_off

        def chunk(j, carry):
            ebase = pl.multiple_of(s * ep + j * CHUNK, 128)
            pltpu.sync_copy(src_r.at[pl.ds(ebase, CHUNK)], src_c)
            pltpu.sync_copy(dst_r.at[pl.ds(ebase, CHUNK)], dst_c)

            if softmax:
                for k in range(CHUNK // 16):
                    sv = src_c[pl.ds(k * 16, 16)]
                    dvx = dst_c[pl.ds(k * 16, 16)]
                    e = (plsc.load_gather(el_t, [sv])
                         + plsc.load_gather(er_t, [dvx]))
                    e = jnp.where(e > 0, e, 0.2 * e)
                    exbuf[pl.ds(k * 16, 16)] = jnp.exp(e - mglob)
            for k in range(CHUNK // 16):
                src_c[pl.ds(k * 16, 16)] = src_c[pl.ds(k * 16, 16)] + coff
            pltpu.async_copy(ztab_r.at[src_c], gbuf, sem).wait()

            if softmax:
                def rowgrp(g, carry2):
                    for u in range(8):
                        r = g * 8 + u
                        exv = plsc.load_gather(
                            exbuf, [jnp.full((16,), r, jnp.int32)])
                        for k in range(8):
                            gbuf[r, pl.ds(k * 16, 16)] = (
                                gbuf[r, pl.ds(k * 16, 16)] * exv)
                    return carry2
                lax.fori_loop(0, CHUNK // 8, rowgrp, 0)
                pltpu.sync_copy(gbuf, accF.at[dst_c], add=True)
                pltpu.sync_copy(exbuf, accD.at[dst_c], add=True)
            else:
                pltpu.sync_copy(gbuf, accF.at[dst_c], add=True)
                if need_den:
                    pltpu.sync_copy(exd, accD.at[dst_c], add=True)
            return carry
        lax.fori_loop(0, nct, chunk, 0)
        plsc.subcore_barrier()

        def drain(b, carry):
            row0 = pl.multiple_of(t0 + b * RB, RB)
            pltpu.sync_copy(accF.at[pl.ds(row0, RB)], gbuf.at[pl.ds(0, RB)])
            if need_den:
                pltpu.sync_copy(accD.at[pl.ds(row0, RB)], denblk)
            for r in range(RB):
                dvv = None
                if need_den:
                    dvv = plsc.load_gather(
                        denblk, [jnp.full((16,), r, jnp.int32)])
                    if softmax:
                        dvv = jnp.where(dvv == 0.0, 1.0, dvv)
                    elif divide:
                        dvv = jnp.maximum(dvv, 1.0)
                for k in range(8):
                    v = gbuf[r, pl.ds(k * 16, 16)]
                    if softmax or divide:
                        v = v / dvv
                    if bias:
                        v = v + bvec[pl.ds(k * 16, 16)]
                    if relu:
                        v = jnp.maximum(v, 0.0)
                    outblk[r, pl.ds(k * 16, 16)] = v
                if want_cnt:
                    plsc.store_scatter(
                        cntbuf, [jnp.full((16,), r, jnp.int32)], dvv)
            orow = pl.multiple_of(c * npad + row0, RB)
            pltpu.sync_copy(outblk, out_r.at[pl.ds(orow, RB)])
            if want_cnt:
                @pl.when(c == 0)
                def _():
                    pltpu.sync_copy(cntbuf, cnt_r.at[pl.ds(row0, RB)])
            return carry
        lax.fori_loop(0, nblk, drain, 0)

    return pl.kernel(body, out_type=out_type, mesh=mesh,
                     scratch_types=scratch,
                     compiler_params=pltpu.CompilerParams(
                         needs_layout_passes=False))


def _prep_edges(src, dst, n_dst):
    e = src.shape[0]
    gran = NT * CHUNK
    e_pad = gran * _cdiv(e, gran)
    if e_pad != e:
        src = jnp.concatenate([src, jnp.zeros((e_pad - e,), jnp.int32)])
        dst = jnp.concatenate([dst, jnp.full((e_pad - e,), n_dst, jnp.int32)])
    return src, dst, e_pad


def _padtab(v, npad):
    return jnp.pad(v, (0, npad - v.shape[0]))


def _gat_mm_body(x_ref, w_ref, al_ref, ar_ref, zlo_ref, zhi_ref, el_ref, er_ref):
    z = jnp.dot(x_ref[...], w_ref[...], preferred_element_type=jnp.float32)
    zlo_ref[...] = z[:, :128]
    zhi_ref[...] = z[:, 128:]
    el_ref[...] = jnp.sum(z * al_ref[...], axis=1, keepdims=True)
    er_ref[...] = jnp.sum(z * ar_ref[...], axis=1, keepdims=True)


def _gat_mm(x, W, al, ar, tn=512):
    n, k = x.shape
    g = W.shape[1]
    grid = (pl.cdiv(n, tn),)
    return pl.pallas_call(
        _gat_mm_body,
        grid=grid,
        in_specs=[
            pl.BlockSpec((tn, k), lambda i: (i, 0)),
            pl.BlockSpec((k, g), lambda i: (0, 0)),
            pl.BlockSpec((1, g), lambda i: (0, 0)),
            pl.BlockSpec((1, g), lambda i: (0, 0)),
        ],
        out_specs=[
            pl.BlockSpec((tn, 128), lambda i: (i, 0)),
            pl.BlockSpec((tn, 128), lambda i: (i, 0)),
            pl.BlockSpec((tn, 1), lambda i: (i, 0)),
            pl.BlockSpec((tn, 1), lambda i: (i, 0)),
        ],
        out_shape=[
            jax.ShapeDtypeStruct((n, 128), jnp.float32),
            jax.ShapeDtypeStruct((n, 128), jnp.float32),
            jax.ShapeDtypeStruct((n, 1), jnp.float32),
            jax.ShapeDtypeStruct((n, 1), jnp.float32),
        ],
    )(x, W, al.reshape(1, g), ar.reshape(1, g))


def _gat_mm2_body(s_ref, x0_ref, cnt_ref, wa_ref, wb_ref, al_ref, ar_ref,
                  zlo_ref, zhi_ref, el_ref, er_ref):
    z = jnp.dot(s_ref[...], wa_ref[...], preferred_element_type=jnp.float32)
    z = z + jnp.dot(x0_ref[...] * cnt_ref[...], wb_ref[...],
                    preferred_element_type=jnp.float32)
    zlo_ref[...] = z[:, :128]
    zhi_ref[...] = z[:, 128:]
    el_ref[...] = jnp.sum(z * al_ref[...], axis=1, keepdims=True)
    er_ref[...] = jnp.sum(z * ar_ref[...], axis=1, keepdims=True)


def _gat_mm2(s, x0, cnt, Wa, Wb, al, ar, tn=512):
    n, k = s.shape
    g = Wa.shape[1]
    grid = (pl.cdiv(n, tn),)
    return pl.pallas_call(
        _gat_mm2_body,
        grid=grid,
        in_specs=[
            pl.BlockSpec((tn, k), lambda i: (i, 0)),
            pl.BlockSpec((tn, k), lambda i: (i, 0)),
            pl.BlockSpec((tn, 1), lambda i: (i, 0)),
            pl.BlockSpec((k, g), lambda i: (0, 0)),
            pl.BlockSpec((k, g), lambda i: (0, 0)),
            pl.BlockSpec((1, g), lambda i: (0, 0)),
            pl.BlockSpec((1, g), lambda i: (0, 0)),
        ],
        out_specs=[
            pl.BlockSpec((tn, 128), lambda i: (i, 0)),
            pl.BlockSpec((tn, 128), lambda i: (i, 0)),
            pl.BlockSpec((tn, 1), lambda i: (i, 0)),
            pl.BlockSpec((tn, 1), lambda i: (i, 0)),
        ],
        out_shape=[
            jax.ShapeDtypeStruct((n, 128), jnp.float32),
            jax.ShapeDtypeStruct((n, 128), jnp.float32),
            jax.ShapeDtypeStruct((n, 1), jnp.float32),
            jax.ShapeDtypeStruct((n, 1), jnp.float32),
        ],
    )(s, x0, cnt.reshape(n, 1), Wa, Wb, al.reshape(1, g), ar.reshape(1, g))


def _head_body(y_ref, w0_ref, b0_ref, w1_ref, b1_ref, o_ref):
    h = jnp.dot(y_ref[...], w0_ref[...], preferred_element_type=jnp.float32)
    h = jnp.maximum(h + b0_ref[...], 0.0)
    o = jnp.dot(h, w1_ref[...], preferred_element_type=jnp.float32)
    o_ref[...] = jnp.maximum(o + b1_ref[...], 0.0)


def _head(y, M0W, M0b, M1W, M1b):
    n, k = y.shape
    h1 = M0W.shape[1]
    h2 = M1W.shape[1]
    return pl.pallas_call(
        _head_body,
        grid=(1,),
        in_specs=[
            pl.BlockSpec((n, k), lambda i: (0, 0)),
            pl.BlockSpec((k, h1), lambda i: (0, 0)),
            pl.BlockSpec((1, h1), lambda i: (0, 0)),
            pl.BlockSpec((h1, h2), lambda i: (0, 0)),
            pl.BlockSpec((1, h2), lambda i: (0, 0)),
        ],
        out_specs=pl.BlockSpec((n, h2), lambda i: (0, 0)),
        out_shape=jax.ShapeDtypeStruct((n, h2), jnp.float32),
    )(y, M0W, M0b.reshape(1, h1), M1W, M1b.reshape(1, h2))


def kernel(X, e00_src, e00_dst, down_src, down_dst, e11_src, e11_dst,
           up_src, up_dst, conn_src, conn_dst, W0, al0, ar0, b0,
           W1, al1, ar1, b1, W2, al2, ar2, b2, M0W, M0b, M1W, M1b):
    npad0 = 1024 * _cdiv(N0 + 1, 1024)
    npad1 = 1024 * _cdiv(N1 + 1, 1024)
    npadn = 1024 * _cdiv(NNET + 1, 1024)

    e00s, e00d, ep00 = _prep_edges(e00_src, e00_dst, N0)
    dns, dnd, epdn = _prep_edges(down_src, down_dst, N1)
    e11s, e11d, ep11 = _prep_edges(e11_src, e11_dst, N1)
    ups, upd, epup = _prep_edges(up_src, up_dst, N0)
    cns, cnd, epcn = _prep_edges(conn_src, conn_dst, NNET)

    z0lo, z0hi, el0, er0 = _gat_mm(X, W0, al0, ar0)
    gat0 = _sc_seg(N0, N0, ep00, True, False, False, True, False)
    (x0f,) = gat0(e00s, e00d, jnp.concatenate([z0lo, z0hi], axis=0),
                  _padtab(el0[:, 0], npad0), _padtab(er0[:, 0], npad0),
                  b0.reshape(2, 128))

    pool_dn = _sc_seg(npad0, N1, epdn, False, True, True, False, False)
    (x1f,) = pool_dn(dns, dnd, x0f)

    x1 = jnp.concatenate([x1f[0:N1], x1f[npad1:npad1 + N1]], axis=1)
    z1lo, z1hi, el1, er1 = _gat_mm(x1, W1, al1, ar1)
    gat1 = _sc_seg(N1, N1, ep11, True, False, True, True, False)
    (x1gf,) = gat1(e11s, e11d, jnp.concatenate([z1lo, z1hi], axis=0),
                   _padtab(el1[:, 0], npad1), _padtab(er1[:, 0], npad1),
                   b1.reshape(2, 128))

    pool_up = _sc_seg(npad1, N0, epup, False, False, False, False, True)
    upf, upcnt = pool_up(ups, upd, x1gf)
    up_sum = jnp.concatenate([upf[0:N0], upf[npad0:npad0 + N0]], axis=1)
    x0cat = jnp.concatenate([x0f[0:N0], x0f[npad0:npad0 + N0]], axis=1)
    z2lo, z2hi, el2, er2 = _gat_mm2(up_sum, x0cat, upcnt[:N0],
                                    W2[:256], W2[256:], al2, ar2)
    gat2 = _sc_seg(N0, N0, ep00, True, False, True, True, False)
    (x0uf,) = gat2(e00s, e00d, jnp.concatenate([z2lo, z2hi], axis=0),
                   _padtab(el2[:, 0], npad0), _padtab(er2[:, 0], npad0),
                   b2.reshape(2, 128))

    pool_cn = _sc_seg(npad0, NNET, epcn, False, False, False, False, False)
    (yf,) = pool_cn(cns, cnd, x0uf)
    y = jnp.concatenate([yf[0:NNET], yf[npadn:npadn + NNET]], axis=1)

    return _head(y, M0W, M0b, M1W, M1b)
